# baseline (speedup 1.0000x reference)
import jax
import jax.numpy as jnp
from jax.experimental import pallas as pl
from jax.experimental.pallas import tpu as pltpu

_CAPACITY = 65536
_EMBED = 256
_PTR = 0
_NFEAT = 4096

_NB = 64
_MROWS = _CAPACITY // _NB
_FROWS = _NFEAT // _NB


def _body(f_ref, m_ref, o_ref, acc_ref):
    i = pl.program_id(0)

    @pl.when(i == 0)
    def _init():
        acc_ref[...] = jnp.zeros_like(acc_ref)

    acc_ref[...] += jnp.sum(f_ref[...], axis=0, keepdims=True)
    o_ref[...] = m_ref[...]

    @pl.when(i == _NB - 1)
    def _finish():
        o_ref[_PTR : _PTR + 1, :] = acc_ref[...] * (1.0 / _NFEAT)


def kernel(feature, memory):
    return pl.pallas_call(
        _body,
        grid=(_NB,),
        in_specs=[
            pl.BlockSpec((_FROWS, _EMBED), lambda i: (i, 0)),
            pl.BlockSpec((_MROWS, _EMBED), lambda i: (_NB - 1 - i, 0)),
        ],
        out_specs=pl.BlockSpec((_MROWS, _EMBED), lambda i: (_NB - 1 - i, 0)),
        out_shape=jax.ShapeDtypeStruct((_CAPACITY, _EMBED), jnp.float32),
        scratch_shapes=[pltpu.VMEM((1, _EMBED), jnp.float32)],
        compiler_params=pltpu.CompilerParams(
            dimension_semantics=("arbitrary",),
        ),
    )(feature, memory)
